# baseline (speedup 1.0000x reference)
import functools

import jax
import jax.numpy as jnp
from jax import lax
from jax.experimental import pallas as pl
from jax.experimental.pallas import tpu as pltpu
from jax.experimental.pallas import tpu_sc as plsc

_CHUNK = 2048


@functools.lru_cache(maxsize=None)
def _make_sc_gather(B, H, ent_rows, rel_rows):
    info = plsc.get_sparse_core_info()
    NC, NS, L = info.num_cores, info.num_subcores, info.num_lanes
    NW = NC * NS
    dims_per_w = H // NW
    n_chunks = B // _CHUNK

    mesh = plsc.VectorSubcoreMesh(core_axis_name="c", subcore_axis_name="s")

    @functools.partial(
        pl.kernel,
        mesh=mesh,
        compiler_params=pltpu.CompilerParams(needs_layout_passes=False),
        out_type=[
            jax.ShapeDtypeStruct((H, B), jnp.float32),
            jax.ShapeDtypeStruct((H, B), jnp.float32),
            jax.ShapeDtypeStruct((H, B), jnp.float32),
        ],
        scratch_types=[
            pltpu.VMEM((ent_rows,), jnp.float32),
            pltpu.VMEM((rel_rows,), jnp.float32),
            pltpu.VMEM((2, _CHUNK), jnp.int32),
            pltpu.VMEM((2, _CHUNK), jnp.int32),
            pltpu.VMEM((2, _CHUNK), jnp.int32),
            pltpu.VMEM((2, _CHUNK), jnp.float32),
            pltpu.VMEM((2, _CHUNK), jnp.float32),
            pltpu.VMEM((2, _CHUNK), jnp.float32),
            pltpu.SemaphoreType.DMA((2,)),
            pltpu.SemaphoreType.DMA((2,)),
            pltpu.VMEM_SHARED((3 * B,), jnp.int32),
        ],
    )
    def sc_k(h_hbm, t_hbm, r_hbm, entT_hbm, relT_hbm,
             ht_out, tt_out, rt_out,
             erow, rrow, hidx, tidx, ridx, hbuf, tbuf, rbuf,
             sem_i, sem_o, sidx):
        wid = lax.axis_index("s") * NC + lax.axis_index("c")
        sid = lax.axis_index("s")
        total_chunks = dims_per_w * n_chunks

        def idx_fetch(c, b):
            base = (c % n_chunks) * _CHUNK
            pltpu.async_copy(sidx.at[pl.ds(base, _CHUNK)], hidx.at[b], sem_i.at[b])
            pltpu.async_copy(sidx.at[pl.ds(B + base, _CHUNK)], tidx.at[b], sem_i.at[b])
            pltpu.async_copy(sidx.at[pl.ds(2 * B + base, _CHUNK)], ridx.at[b], sem_i.at[b])

        def idx_wait(b):
            sl = pl.ds(0, _CHUNK)
            pltpu.make_async_copy(sidx.at[sl], hidx.at[b], sem_i.at[b]).wait()
            pltpu.make_async_copy(sidx.at[sl], tidx.at[b], sem_i.at[b]).wait()
            pltpu.make_async_copy(sidx.at[sl], ridx.at[b], sem_i.at[b]).wait()

        def out_write(c, b):
            j = wid * dims_per_w + c // n_chunks
            base = (c % n_chunks) * _CHUNK
            sl = pl.ds(base, _CHUNK)
            pltpu.async_copy(hbuf.at[b], ht_out.at[j, sl], sem_o.at[b])
            pltpu.async_copy(tbuf.at[b], tt_out.at[j, sl], sem_o.at[b])
            pltpu.async_copy(rbuf.at[b], rt_out.at[j, sl], sem_o.at[b])

        def out_wait(b):
            j0 = wid * dims_per_w
            sl = pl.ds(0, _CHUNK)
            pltpu.make_async_copy(hbuf.at[b], ht_out.at[j0, sl], sem_o.at[b]).wait()
            pltpu.make_async_copy(tbuf.at[b], tt_out.at[j0, sl], sem_o.at[b]).wait()
            pltpu.make_async_copy(rbuf.at[b], rt_out.at[j0, sl], sem_o.at[b]).wait()

        @pl.when(sid == 0)
        def _():
            pltpu.sync_copy(h_hbm, sidx.at[pl.ds(0, B)])

        @pl.when(sid == 1)
        def _():
            pltpu.sync_copy(t_hbm, sidx.at[pl.ds(B, B)])

        @pl.when(sid == 2)
        def _():
            pltpu.sync_copy(r_hbm, sidx.at[pl.ds(2 * B, B)])

        pltpu.sync_copy(relT_hbm.at[wid * dims_per_w], rrow)
        pltpu.sync_copy(entT_hbm.at[wid * dims_per_w], erow)
        plsc.subcore_barrier()
        idx_fetch(0, 0)

        def outer(c0, carry):
            for b in range(2):
                c = c0 * 2 + b
                nxt = c + 1

                @pl.when(nxt < total_chunks)
                def _():
                    idx_fetch(nxt, 1 - b)

                @pl.when((c > 0) & (c % n_chunks == 0))
                def _():
                    j = wid * dims_per_w + c // n_chunks
                    pltpu.sync_copy(entT_hbm.at[j], erow)
                    pltpu.sync_copy(relT_hbm.at[j], rrow)

                idx_wait(b)

                @pl.when(c >= 2)
                def _():
                    out_wait(b)

                bb = b

                @plsc.parallel_loop(0, _CHUNK, L, unroll=32)
                def vec(v):
                    sl = pl.ds(v, L)
                    hbuf[bb, sl] = plsc.load_gather(erow, [hidx[bb, sl]])
                    tbuf[bb, sl] = plsc.load_gather(erow, [tidx[bb, sl]])
                    rbuf[bb, sl] = plsc.load_gather(rrow, [ridx[bb, sl]])

                out_write(c, b)
            return carry

        lax.fori_loop(0, total_chunks // 2, outer, 0)
        out_wait(0)
        out_wait(1)

    return sc_k


def _tc_norm_body(h_raw_ref, t_raw_ref, hidx_ref, tidx_ref,
                  h_out_ref, t_out_ref, *, last_row):
    for raw_ref, idx_ref, out_ref in (
        (h_raw_ref, hidx_ref, h_out_ref),
        (t_raw_ref, tidx_ref, t_out_ref),
    ):
        x = raw_ref[...]
        idx = idx_ref[...]
        ss = jnp.sum(x * x, axis=0, keepdims=True)
        norm = jnp.sqrt(ss)
        keep_raw = idx == last_row
        scale = jnp.where(keep_raw, jnp.ones_like(norm), 1.0 / norm)
        out_ref[...] = x * scale


@functools.lru_cache(maxsize=None)
def _make_tc_norm(B, H, ent_rows, blk=16384):
    return pl.pallas_call(
        functools.partial(_tc_norm_body, last_row=ent_rows - 1),
        grid=(B // blk,),
        in_specs=[
            pl.BlockSpec((H, blk), lambda i: (0, i)),
            pl.BlockSpec((H, blk), lambda i: (0, i)),
            pl.BlockSpec((1, blk), lambda i: (0, i)),
            pl.BlockSpec((1, blk), lambda i: (0, i)),
        ],
        out_specs=[
            pl.BlockSpec((H, blk), lambda i: (0, i)),
            pl.BlockSpec((H, blk), lambda i: (0, i)),
        ],
        out_shape=[
            jax.ShapeDtypeStruct((H, B), jnp.float32),
            jax.ShapeDtypeStruct((H, B), jnp.float32),
        ],
    )


def kernel(h, r, t, ent_table, rel_table, type_table):
    B = h.shape[0]
    ent_rows, H = ent_table.shape
    rel_rows = rel_table.shape[0]
    h32 = h.astype(jnp.int32)
    t32 = t.astype(jnp.int32)
    r32 = r.astype(jnp.int32)
    entT = ent_table.T
    relT = rel_table.T
    sc_k = _make_sc_gather(B, H, ent_rows, rel_rows)
    h_rawT, t_rawT, r_embT = sc_k(h32, t32, r32, entT, relT)
    tc_norm = _make_tc_norm(B, H, ent_rows)
    h_embT, t_embT = tc_norm(h_rawT, t_rawT,
                             h32.reshape(1, B), t32.reshape(1, B))
    return (h_embT.T, t_embT.T, r_embT.T)


# speedup vs baseline: 1.0550x; 1.0550x over previous
import functools

import jax
import jax.numpy as jnp
from jax import lax
from jax.experimental import pallas as pl
from jax.experimental.pallas import tpu as pltpu
from jax.experimental.pallas import tpu_sc as plsc

_CHUNK = 2048


@functools.lru_cache(maxsize=None)
def _make_sc_gather(B, H, ent_rows, rel_rows):
    info = plsc.get_sparse_core_info()
    NC, NS, L = info.num_cores, info.num_subcores, info.num_lanes
    NW = NC * NS
    dims_per_w = H // NW
    n_chunks = B // _CHUNK

    mesh = plsc.VectorSubcoreMesh(core_axis_name="c", subcore_axis_name="s")

    @functools.partial(
        pl.kernel,
        mesh=mesh,
        compiler_params=pltpu.CompilerParams(needs_layout_passes=False),
        out_type=[
            jax.ShapeDtypeStruct((H, B), jnp.float32),
            jax.ShapeDtypeStruct((H, B), jnp.float32),
            jax.ShapeDtypeStruct((H, B), jnp.float32),
        ],
        scratch_types=[
            pltpu.VMEM((ent_rows,), jnp.float32),
            pltpu.VMEM((rel_rows,), jnp.float32),
            pltpu.VMEM((2, _CHUNK), jnp.int32),
            pltpu.VMEM((2, _CHUNK), jnp.int32),
            pltpu.VMEM((2, _CHUNK), jnp.int32),
            pltpu.VMEM((2, _CHUNK), jnp.float32),
            pltpu.VMEM((2, _CHUNK), jnp.float32),
            pltpu.VMEM((2, _CHUNK), jnp.float32),
            pltpu.SemaphoreType.DMA((2,)),
            pltpu.SemaphoreType.DMA((2,)),
            pltpu.SemaphoreType.DMA,
            pltpu.VMEM_SHARED((3 * B,), jnp.int32),
        ],
    )
    def sc_k(h_hbm, t_hbm, r_hbm, entT_hbm, relT_hbm,
             ht_out, tt_out, rt_out,
             erow, rrow, hidx, tidx, ridx, hbuf, tbuf, rbuf,
             sem_i, sem_o, sem_e, sidx):
        wid = lax.axis_index("s") * NC + lax.axis_index("c")
        sid = lax.axis_index("s")
        total_chunks = dims_per_w * n_chunks

        def idx_fetch(c, b):
            base = (c % n_chunks) * _CHUNK
            pltpu.async_copy(sidx.at[pl.ds(base, _CHUNK)], hidx.at[b], sem_i.at[b])
            pltpu.async_copy(sidx.at[pl.ds(B + base, _CHUNK)], tidx.at[b], sem_i.at[b])
            pltpu.async_copy(sidx.at[pl.ds(2 * B + base, _CHUNK)], ridx.at[b], sem_i.at[b])

        def idx_wait(b):
            sl = pl.ds(0, _CHUNK)
            pltpu.make_async_copy(sidx.at[sl], hidx.at[b], sem_i.at[b]).wait()
            pltpu.make_async_copy(sidx.at[sl], tidx.at[b], sem_i.at[b]).wait()
            pltpu.make_async_copy(sidx.at[sl], ridx.at[b], sem_i.at[b]).wait()

        def out_write(c, b):
            j = wid * dims_per_w + c // n_chunks
            base = (c % n_chunks) * _CHUNK
            sl = pl.ds(base, _CHUNK)
            pltpu.async_copy(hbuf.at[b], ht_out.at[j, sl], sem_o.at[b])
            pltpu.async_copy(tbuf.at[b], tt_out.at[j, sl], sem_o.at[b])
            pltpu.async_copy(rbuf.at[b], rt_out.at[j, sl], sem_o.at[b])

        def out_wait(b):
            j0 = wid * dims_per_w
            sl = pl.ds(0, _CHUNK)
            pltpu.make_async_copy(hbuf.at[b], ht_out.at[j0, sl], sem_o.at[b]).wait()
            pltpu.make_async_copy(tbuf.at[b], tt_out.at[j0, sl], sem_o.at[b]).wait()
            pltpu.make_async_copy(rbuf.at[b], rt_out.at[j0, sl], sem_o.at[b]).wait()

        pltpu.async_copy(entT_hbm.at[wid * dims_per_w], erow, sem_e)
        seg = B // NS
        off = sid * seg
        pltpu.sync_copy(h_hbm.at[pl.ds(off, seg)], sidx.at[pl.ds(off, seg)])
        pltpu.sync_copy(t_hbm.at[pl.ds(off, seg)], sidx.at[pl.ds(B + off, seg)])
        pltpu.sync_copy(r_hbm.at[pl.ds(off, seg)], sidx.at[pl.ds(2 * B + off, seg)])
        pltpu.sync_copy(relT_hbm.at[wid * dims_per_w], rrow)
        plsc.subcore_barrier()
        idx_fetch(0, 0)
        pltpu.make_async_copy(entT_hbm.at[wid * dims_per_w], erow, sem_e).wait()

        def outer(c0, carry):
            for b in range(2):
                c = c0 * 2 + b
                nxt = c + 1

                @pl.when(nxt < total_chunks)
                def _():
                    idx_fetch(nxt, 1 - b)

                @pl.when((c > 0) & (c % n_chunks == 0))
                def _():
                    j = wid * dims_per_w + c // n_chunks
                    pltpu.sync_copy(entT_hbm.at[j], erow)
                    pltpu.sync_copy(relT_hbm.at[j], rrow)

                idx_wait(b)

                @pl.when(c >= 2)
                def _():
                    out_wait(b)

                bb = b

                @plsc.parallel_loop(0, _CHUNK, L, unroll=16)
                def vec(v):
                    sl = pl.ds(v, L)
                    hbuf[bb, sl] = plsc.load_gather(erow, [hidx[bb, sl]])
                    tbuf[bb, sl] = plsc.load_gather(erow, [tidx[bb, sl]])
                    rbuf[bb, sl] = plsc.load_gather(rrow, [ridx[bb, sl]])

                out_write(c, b)
            return carry

        lax.fori_loop(0, total_chunks // 2, outer, 0)
        out_wait(0)
        out_wait(1)

    return sc_k


def _tc_norm_body(h_raw_ref, t_raw_ref, hidx_ref, tidx_ref,
                  h_out_ref, t_out_ref, *, last_row):
    for raw_ref, idx_ref, out_ref in (
        (h_raw_ref, hidx_ref, h_out_ref),
        (t_raw_ref, tidx_ref, t_out_ref),
    ):
        x = raw_ref[...]
        idx = idx_ref[...]
        ss = jnp.sum(x * x, axis=0, keepdims=True)
        norm = jnp.sqrt(ss)
        keep_raw = idx == last_row
        scale = jnp.where(keep_raw, jnp.ones_like(norm), 1.0 / norm)
        out_ref[...] = x * scale


@functools.lru_cache(maxsize=None)
def _make_tc_norm(B, H, ent_rows, blk=16384):
    return pl.pallas_call(
        functools.partial(_tc_norm_body, last_row=ent_rows - 1),
        grid=(B // blk,),
        in_specs=[
            pl.BlockSpec((H, blk), lambda i: (0, i)),
            pl.BlockSpec((H, blk), lambda i: (0, i)),
            pl.BlockSpec((1, blk), lambda i: (0, i)),
            pl.BlockSpec((1, blk), lambda i: (0, i)),
        ],
        out_specs=[
            pl.BlockSpec((H, blk), lambda i: (0, i)),
            pl.BlockSpec((H, blk), lambda i: (0, i)),
        ],
        out_shape=[
            jax.ShapeDtypeStruct((H, B), jnp.float32),
            jax.ShapeDtypeStruct((H, B), jnp.float32),
        ],
    )


def kernel(h, r, t, ent_table, rel_table, type_table):
    B = h.shape[0]
    ent_rows, H = ent_table.shape
    rel_rows = rel_table.shape[0]
    h32 = h.astype(jnp.int32)
    t32 = t.astype(jnp.int32)
    r32 = r.astype(jnp.int32)
    entT = ent_table.T
    relT = rel_table.T
    sc_k = _make_sc_gather(B, H, ent_rows, rel_rows)
    h_rawT, t_rawT, r_embT = sc_k(h32, t32, r32, entT, relT)
    tc_norm = _make_tc_norm(B, H, ent_rows)
    h_embT, t_embT = tc_norm(h_rawT, t_rawT,
                             h32.reshape(1, B), t32.reshape(1, B))
    return (h_embT.T, t_embT.T, r_embT.T)
